# baseline (speedup 1.0000x reference)
import functools

import jax
import jax.numpy as jnp
from jax import lax
from jax.experimental import pallas as pl
from jax.experimental.pallas import tpu as pltpu
from jax.experimental.pallas import tpu_sc as plsc

N = 10000
E = 160000
D = 128
DS = 16
DE = 16
H = 8
K = 2
WIDTH = 2 * D
MLP_IN = 2 * D + 2 * DS + DE
GW = 3 * D
HS = D // H

NCORE = 2
NSUB = 16
NWORK = NCORE * NSUB
CH = 128
EP = 163840
NCHUNK = EP // CH
CPW = NCHUNK // NWORK
NA = 10240
RPS = NA // NSUB
F32 = jnp.float32

def _mesh():
    return plsc.VectorSubcoreMesh(core_axis_name="c", subcore_axis_name="s",
                                  num_cores=NCORE, num_subcores=NSUB)


def _zero_vmem(buf, cols):
    z = jnp.zeros((16,), F32)

    @pl.loop(0, buf.shape[0])
    def _(i):
        for j in range(cols // 16):
            buf[i, pl.ds(j * 16, 16)] = z


NB = 4
NBS = 2


def _sc_gather_multi(tables, idxs):
    T = len(idxs)

    @functools.partial(
        pl.kernel,
        out_type=[jax.ShapeDtypeStruct((EP, D), F32) for _ in range(T)],
        mesh=_mesh(),
        scratch_types=[
            pltpu.VMEM((CPW, CH), jnp.int32),
            pltpu.VMEM((NB, CH, D), F32),
            pltpu.SemaphoreType.DMA,
        ],
    )
    def k(*refs):
        tabs = refs[:T]
        idx_h = refs[T:2 * T]
        outs = refs[2 * T:3 * T]
        slab, rows, sem = refs[3 * T:]
        wid = lax.axis_index("s") * NCORE + lax.axis_index("c")
        c0 = wid * CPW

        for t in range(T):
            pltpu.sync_copy(idx_h[t].at[pl.ds(c0, CPW)], slab)

            @pl.loop(0, CPW // NB)
            def _(jo):
                cs = [
                    pltpu.async_copy(
                        tabs[t].at[slab.at[jo * NB + b]], rows.at[b], sem)
                    for b in range(NB)
                ]
                for b in range(NB):
                    cs[b].wait()
                    pltpu.sync_copy(
                        rows.at[b],
                        outs[t].at[pl.ds((c0 + jo * NB + b) * CH, CH)])

    return k(*tables, *idxs)


def _sc_scatter_add(vals, idx):
    @functools.partial(
        pl.kernel,
        out_type=jax.ShapeDtypeStruct((NCORE * NA, D), F32),
        mesh=_mesh(),
        scratch_types=[
            pltpu.VMEM((CPW, CH), jnp.int32),
            pltpu.VMEM((NBS, CH, D), F32),
            pltpu.VMEM_SHARED((NA, D), F32),
            pltpu.SemaphoreType.DMA,
        ],
    )
    def k(vals_h, idx_h, out_o, slab, vbuf, acc, sem):
        cid = lax.axis_index("c")
        sid = lax.axis_index("s")
        c0 = cid * (NCHUNK // NCORE) + sid * CPW

        _zero_vmem(vbuf.at[0], D)
        for b in range(RPS // CH):
            pltpu.sync_copy(vbuf.at[0], acc.at[pl.ds(sid * RPS + b * CH, CH)])
        pltpu.sync_copy(idx_h.at[pl.ds(c0, CPW)], slab)
        plsc.subcore_barrier()

        @pl.loop(0, CPW // NBS)
        def _(jo):
            cs = [
                pltpu.async_copy(
                    vals_h.at[pl.ds((c0 + jo * NBS + b) * CH, CH)],
                    vbuf.at[b], sem)
                for b in range(NBS)
            ]
            for b in range(NBS):
                cs[b].wait()
                pltpu.sync_copy(vbuf.at[b], acc.at[slab.at[jo * NBS + b]],
                                add=True)

        plsc.subcore_barrier()
        for b in range(RPS // CH):
            r = sid * RPS + b * CH
            pltpu.sync_copy(acc.at[pl.ds(r, CH)], vbuf.at[0])
            pltpu.sync_copy(vbuf.at[0], out_o.at[pl.ds(cid * NA + r, CH)])

    return k(vals, idx).reshape(NCORE, NA, D)


CPW2 = NCHUNK // NSUB


def _sc_scatter_add2(vals0, vals1, idx):
    @functools.partial(
        pl.kernel,
        out_type=jax.ShapeDtypeStruct((NCORE * NA, D), F32),
        mesh=_mesh(),
        scratch_types=[
            pltpu.VMEM((CPW2, CH), jnp.int32),
            pltpu.VMEM((NBS, CH, D), F32),
            pltpu.VMEM_SHARED((NA, D), F32),
            pltpu.SemaphoreType.DMA,
        ],
    )
    def k(v0_h, v1_h, idx_h, out_o, slab, vbuf, acc, sem):
        cid = lax.axis_index("c")
        sid = lax.axis_index("s")
        c0 = sid * CPW2

        _zero_vmem(vbuf.at[0], D)
        for b in range(RPS // CH):
            pltpu.sync_copy(vbuf.at[0], acc.at[pl.ds(sid * RPS + b * CH, CH)])
        pltpu.sync_copy(idx_h.at[pl.ds(c0, CPW2)], slab)
        plsc.subcore_barrier()

        def edge_loop(vals_h):
            @pl.loop(0, CPW2 // NBS)
            def _(jo):
                cs = [
                    pltpu.async_copy(
                        vals_h.at[pl.ds((c0 + jo * NBS + b) * CH, CH)],
                        vbuf.at[b], sem)
                    for b in range(NBS)
                ]
                for b in range(NBS):
                    cs[b].wait()
                    pltpu.sync_copy(vbuf.at[b],
                                    acc.at[slab.at[jo * NBS + b]], add=True)

        @pl.when(cid == 0)
        def _():
            edge_loop(v0_h)

        @pl.when(cid == 1)
        def _():
            edge_loop(v1_h)

        plsc.subcore_barrier()
        for b in range(RPS // CH):
            r = sid * RPS + b * CH
            pltpu.sync_copy(acc.at[pl.ds(r, CH)], vbuf.at[0])
            pltpu.sync_copy(vbuf.at[0], out_o.at[pl.ds(cid * NA + r, CH)])

    out = k(vals0, vals1, idx)
    return out[:NA], out[NA:]


BN = 2000
BEB = 2048


def _dot16(a, b):
    return jnp.dot(a.astype(jnp.bfloat16), b.astype(jnp.bfloat16),
                   preferred_element_type=F32)


def _head_expand():
    lane = lax.broadcasted_iota(jnp.int32, (H, D), 1)
    row = lax.broadcasted_iota(jnp.int32, (H, D), 0)
    return (lane // HS == row).astype(F32)


def _ln_math(h, g, b):
    m = jnp.mean(h, axis=-1, keepdims=True)
    v = jnp.mean((h - m) ** 2, axis=-1, keepdims=True)
    return (h - m) * jax.lax.rsqrt(v + 1e-5) * g + b


def _tc_ln(h, lng, lnb):
    def body(g_ref, b_ref, h_ref, o_ref):
        o_ref[...] = _ln_math(h_ref[...], g_ref[...], b_ref[...])

    return pl.pallas_call(
        body,
        grid=(N // BN,),
        in_specs=[
            pl.BlockSpec((1, D), lambda i: (0, 0)),
            pl.BlockSpec((1, D), lambda i: (0, 0)),
            pl.BlockSpec((BN, D), lambda i: (i, 0)),
        ],
        out_specs=pl.BlockSpec((BN, D), lambda i: (i, 0)),
        out_shape=jax.ShapeDtypeStruct((N, D), F32),
    )(lng, lnb, h)


def _tc_edge_mlp(fwd, hs, hd, xsa, xsb, ef, at, bt, c1, c2, c3, w2t):
    def body(hs_ref, hd_ref, xsa_ref, xsb_ref, ef_ref,
             at_ref, bt_ref, c1_ref, c2_ref, c3_ref, w2_ref,
             gate_ref, msg_ref):
        hsv = hs_ref[...]
        pre = _dot16(hsv, at_ref[...])
        pre += _dot16(hd_ref[...], bt_ref[...])
        pre += _dot16(xsa_ref[...], c1_ref[...])
        pre += _dot16(xsb_ref[...], c2_ref[...])
        pre += _dot16(ef_ref[...], c3_ref[...])
        hid = jnp.maximum(pre, 0.0)
        raw = _dot16(hid, w2_ref[...])
        if fwd:
            raw = jnp.where(raw >= 0.0, raw, 0.01 * raw)
            gate = jnp.exp(raw * (1.0 / (HS ** 0.5)))
        else:
            gate = jax.nn.sigmoid(raw)
        gb = jnp.dot(gate[:, :H], _head_expand(),
                     preferred_element_type=F32,
                     precision=lax.Precision.HIGHEST)
        gate_ref[...] = gb if fwd else gate
        msg_ref[...] = hsv * gb

    gate_cols = D if fwd else 2 * H
    wspec = lambda shp: pl.BlockSpec(shp, lambda i: (0, 0))
    return pl.pallas_call(
        body,
        grid=(EP // BEB,),
        in_specs=[
            pl.BlockSpec((BEB, D), lambda i: (i, 0)),
            pl.BlockSpec((BEB, D), lambda i: (i, 0)),
            pl.BlockSpec((BEB, DS), lambda i: (i, 0)),
            pl.BlockSpec((BEB, DS), lambda i: (i, 0)),
            pl.BlockSpec((BEB, DE), lambda i: (i, 0)),
            wspec((D, WIDTH)), wspec((D, WIDTH)),
            wspec((DS, WIDTH)), wspec((DS, WIDTH)), wspec((DE, WIDTH)),
            wspec((WIDTH, 2 * H)),
        ],
        out_specs=[
            pl.BlockSpec((BEB, gate_cols), lambda i: (i, 0)),
            pl.BlockSpec((BEB, D), lambda i: (i, 0)),
        ],
        out_shape=[
            jax.ShapeDtypeStruct((EP, gate_cols), F32),
            jax.ShapeDtypeStruct((EP, D), F32),
        ],
    )(hs, hd, xsa, xsb, ef, at, bt, c1, c2, c3, w2t)


def _tc_edge_mlp2(hsf, hdf, hsr, hdr, xs_src, xs_dest, ef, wf, wr):
    def body(hsf_ref, hdf_ref, hsr_ref, hdr_ref, xss_ref, xsd_ref, ef_ref,
             fat, fbt, fc1, fc2, fc3, fw2,
             rat, rbt, rc1, rc2, rc3, rw2,
             gbf_ref, msgf_ref, gr_ref, msgr_ref):
        efv = ef_ref[...]

        def mlp(hs_v, hd_v, xa_v, xb_v, ws):
            at, bt, c1, c2, c3, w2 = ws
            pre = _dot16(hs_v, at[...])
            pre += _dot16(hd_v, bt[...])
            pre += _dot16(xa_v, c1[...])
            pre += _dot16(xb_v, c2[...])
            pre += _dot16(efv, c3[...])
            return _dot16(jnp.maximum(pre, 0.0), w2[...])

        hsfv = hsf_ref[...]
        raw = mlp(hsfv, hdf_ref[...], xss_ref[...], xsd_ref[...],
                  (fat, fbt, fc1, fc2, fc3, fw2))
        raw = jnp.where(raw >= 0.0, raw, 0.01 * raw)
        gate = jnp.exp(raw * (1.0 / (HS ** 0.5)))
        gb = jnp.dot(gate[:, :H], _head_expand(),
                     preferred_element_type=F32,
                     precision=lax.Precision.HIGHEST)
        gbf_ref[...] = gb
        msgf_ref[...] = hsfv * gb

        hsrv = hsr_ref[...]
        rraw = mlp(hsrv, hdr_ref[...], xsd_ref[...], xss_ref[...],
                   (rat, rbt, rc1, rc2, rc3, rw2))
        g = jax.nn.sigmoid(rraw)
        gr_ref[...] = g
        gbr = jnp.dot(g[:, :H], _head_expand(),
                      preferred_element_type=F32,
                      precision=lax.Precision.HIGHEST)
        msgr_ref[...] = hsrv * gbr

    espec = lambda c: pl.BlockSpec((BEB, c), lambda i: (i, 0))
    wspec = lambda shp: pl.BlockSpec(shp, lambda i: (0, 0))
    wspecs = [wspec((D, WIDTH)), wspec((D, WIDTH)), wspec((DS, WIDTH)),
              wspec((DS, WIDTH)), wspec((DE, WIDTH)), wspec((WIDTH, 2 * H))]
    return pl.pallas_call(
        body,
        grid=(EP // BEB,),
        in_specs=[espec(D)] * 4 + [espec(DS), espec(DS), espec(DE)]
        + wspecs + wspecs,
        out_specs=[espec(D), espec(D), espec(2 * H), espec(D)],
        out_shape=[
            jax.ShapeDtypeStruct((EP, D), F32),
            jax.ShapeDtypeStruct((EP, D), F32),
            jax.ShapeDtypeStruct((EP, 2 * H), F32),
            jax.ShapeDtypeStruct((EP, D), F32),
        ],
    )(hsf, hdf, hsr, hdr, xs_src, xs_dest, ef, *wf, *wr)


def _tc_update(h_prev, agg, pt, lng, lnb, smb=None):
    with_sm = smb is not None

    def body(*refs):
        if with_sm:
            g_ref, b_ref, h_ref, a_ref, s_ref, p_ref, hn_ref, ln_ref = refs
            agg_v = a_ref[...]
            smv = s_ref[...]
            agg_v = agg_v / jnp.where(smv > 0.0, smv, 1.0)
        else:
            g_ref, b_ref, h_ref, a_ref, p_ref, hn_ref, ln_ref = refs
            agg_v = a_ref[0] + a_ref[1]
        m = _dot16(agg_v, p_ref[...])
        hn = h_ref[...] + m
        hn_ref[...] = hn
        ln_ref[...] = _ln_math(hn, g_ref[...], b_ref[...])

    nspec = pl.BlockSpec((BN, D), lambda i: (i, 0))
    in_specs = [
        pl.BlockSpec((1, D), lambda i: (0, 0)),
        pl.BlockSpec((1, D), lambda i: (0, 0)),
        nspec,
    ]
    args = [lng, lnb, h_prev]
    if with_sm:
        in_specs += [nspec, nspec]
        args += [agg, smb]
    else:
        in_specs.append(pl.BlockSpec((NCORE, BN, D), lambda i: (0, i, 0)))
        args.append(agg)
    in_specs.append(pl.BlockSpec((D, D), lambda i: (0, 0)))
    args.append(pt)
    return pl.pallas_call(
        body,
        grid=(N // BN,),
        in_specs=in_specs,
        out_specs=[nspec, nspec],
        out_shape=[
            jax.ShapeDtypeStruct((N, D), F32),
            jax.ShapeDtypeStruct((N, D), F32),
        ],
    )(*args)


def _tc_wdiv(gb1, gb2, smg1, smg2):
    def body(e1, e2, s1, s2, w1, w2):
        comp = _head_expand().T * (1.0 / HS)
        for e, s_, w in ((e1, s1, w1), (e2, s2, w2)):
            wf = e[...] / (s_[...] + 1e-9)
            w[...] = jnp.dot(wf, comp, preferred_element_type=F32,
                             precision=lax.Precision.HIGHEST)

    return pl.pallas_call(
        body,
        grid=(EP // BEB,),
        in_specs=[pl.BlockSpec((BEB, D), lambda i: (i, 0))] * 4,
        out_specs=[pl.BlockSpec((BEB, H), lambda i: (i, 0))] * 2,
        out_shape=[jax.ShapeDtypeStruct((EP, H), F32)] * 2,
    )(gb1, gb2, smg1, smg2)


def _tc_gating(x, hf, hr, rw1t, rb1, rw2t, rb2, uw1t, ub1, uw2t, ub2,
               cw1t, cb1, cw2t, cb2):
    def body(x_ref, hf_ref, hr_ref,
             rw1_ref, rb1_ref, rw2_ref, rb2_ref,
             uw1_ref, ub1_ref, uw2_ref, ub2_ref,
             cw1_ref, cb1_ref, cw2_ref, cb2_ref,
             fin_ref, z_ref, r_ref):
        xv = x_ref[...]
        mf = hf_ref[...] - xv
        mr = hr_ref[...] - xv

        def mlp2(a0, w1_ref, b1_ref, w2_ref, b2_ref):
            h1 = _dot16(a0, w1_ref[pl.ds(0, D), :])
            h1 += _dot16(mf, w1_ref[pl.ds(D, D), :])
            h1 += _dot16(mr, w1_ref[pl.ds(2 * D, D), :])
            h1 = jnp.maximum(h1 + b1_ref[...], 0.0)
            return _dot16(h1, w2_ref[...]) + b2_ref[...]

        r = jax.nn.sigmoid(mlp2(xv, rw1_ref, rb1_ref, rw2_ref, rb2_ref))
        z = jax.nn.sigmoid(mlp2(xv, uw1_ref, ub1_ref, uw2_ref, ub2_ref))
        c = jnp.tanh(mlp2(r * xv, cw1_ref, cb1_ref, cw2_ref, cb2_ref))
        fin_ref[...] = (1.0 - z) * xv + z * c
        z_ref[...] = z
        r_ref[...] = r

    nspec = pl.BlockSpec((BN, D), lambda i: (i, 0))
    w1spec = pl.BlockSpec((GW, GW), lambda i: (0, 0))
    b1spec = pl.BlockSpec((1, GW), lambda i: (0, 0))
    w2spec = pl.BlockSpec((GW, D), lambda i: (0, 0))
    b2spec = pl.BlockSpec((1, D), lambda i: (0, 0))
    return pl.pallas_call(
        body,
        grid=(N // BN,),
        in_specs=[nspec, nspec, nspec] + [w1spec, b1spec, w2spec, b2spec] * 3,
        out_specs=[nspec, nspec, nspec],
        out_shape=[jax.ShapeDtypeStruct((N, D), F32)] * 3,
    )(x, hf, hr, rw1t, rb1, rw2t, rb2, uw1t, ub1, uw2t, ub2,
      cw1t, cb1, cw2t, cb2)


def _pad_idx(a, pad_val):
    pad = jnp.full((EP - E,), pad_val, jnp.int32)
    return jnp.concatenate([a, pad]).reshape(NCHUNK, CH)


def kernel(x, x_s, edge_index, edge_features, fw1, fw2, fp, rw1, rw2, rp,
           lng, lnb, rg_w1, rg_b1, rg_w2, rg_b2, ug_w1, ug_b1, ug_w2, ug_b2,
           cd_w1, cd_b1, cd_w2, cd_b2):
    src = edge_index[0]
    dest = edge_index[1]
    src_g = _pad_idx(src, 0)
    dest_g = _pad_idx(dest, 0)
    src_s = _pad_idx(src, N)
    dest_s = _pad_idx(dest, N)
    ef_p = jnp.concatenate(
        [edge_features, jnp.zeros((EP - E, DE), F32)], axis=0)

    lng2 = lng.reshape(1, D)
    lnb2 = lnb.reshape(1, D)

    fw1t = fw1.transpose(0, 2, 1)
    rw1t = rw1.transpose(0, 2, 1)
    zpad = jnp.zeros((K, WIDTH, H), F32)
    fw2t = jnp.concatenate([fw2.transpose(0, 2, 1), zpad], axis=-1)
    rw2t = jnp.concatenate([rw2.transpose(0, 2, 1), zpad], axis=-1)
    fpt = fp.transpose(0, 2, 1)
    rpt = rp.transpose(0, 2, 1)

    def wsplit(w1t, i):
        return (w1t[i, :D], w1t[i, D:2 * D], w1t[i, 2 * D:2 * D + DS],
                w1t[i, 2 * D + DS:2 * D + 2 * DS], w1t[i, 2 * D + 2 * DS:])

    ln0 = _tc_ln(x, lng2, lnb2)

    xs_pad = jnp.concatenate([x_s, jnp.zeros((N, D - DS), F32)], axis=1)
    gx_src, gx_dest, hs1, hd1 = _sc_gather_multi(
        [xs_pad, xs_pad, ln0, ln0], [src_g, dest_g, src_g, dest_g])
    xs_src = gx_src[:, :DS]
    xs_dest = gx_dest[:, :DS]

    gb1, msgf1, g1, msgr1 = _tc_edge_mlp2(
        hs1, hd1, hd1, hs1, xs_src, xs_dest, ef_p,
        (*wsplit(fw1t, 0), fw2t[0]), (*wsplit(rw1t, 0), rw2t[0]))
    agg1, sm1 = _sc_scatter_add2(msgf1, gb1, dest_s)
    aggr1 = _sc_scatter_add(msgr1, src_s)
    h_f1, ln_f1 = _tc_update(x, agg1, fpt[0], lng2, lnb2, smb=sm1)
    h_r1, ln_r1 = _tc_update(x, aggr1, rpt[0], lng2, lnb2)

    hs2, hd2, hs2r, hd2r, smg1 = _sc_gather_multi(
        [ln_f1, ln_f1, ln_r1, ln_r1, sm1],
        [src_g, dest_g, dest_g, src_g, dest_g])
    gb2, msgf2, g2, msgr2 = _tc_edge_mlp2(
        hs2, hd2, hs2r, hd2r, xs_src, xs_dest, ef_p,
        (*wsplit(fw1t, 1), fw2t[1]), (*wsplit(rw1t, 1), rw2t[1]))
    agg2, sm2 = _sc_scatter_add2(msgf2, gb2, dest_s)
    aggr2 = _sc_scatter_add(msgr2, src_s)
    h_f, _ = _tc_update(h_f1, agg2, fpt[1], lng2, lnb2, smb=sm2)
    h_r, _ = _tc_update(h_r1, aggr2, rpt[1], lng2, lnb2)
    gs = [g1, g2]

    (smg2,) = _sc_gather_multi([sm2], [dest_g])
    w1, w2 = _tc_wdiv(gb1, gb2, smg1, smg2)

    final, z, r = _tc_gating(
        x, h_f, h_r,
        rg_w1.T, rg_b1.reshape(1, GW), rg_w2.T, rg_b2.reshape(1, D),
        ug_w1.T, ug_b1.reshape(1, GW), ug_w2.T, ug_b2.reshape(1, D),
        cd_w1.T, cd_b1.reshape(1, GW), cd_w2.T, cd_b2.reshape(1, D))

    fws = jnp.stack([w1[:E], w2[:E]], axis=-1)
    rws = jnp.stack([gs[0][:E, :H], gs[1][:E, :H]], axis=-1)
    return final, fws, rws, z, r


# speedup vs baseline: 1.0871x; 1.0871x over previous
import functools

import jax
import jax.numpy as jnp
from jax import lax
from jax.experimental import pallas as pl
from jax.experimental.pallas import tpu as pltpu
from jax.experimental.pallas import tpu_sc as plsc

N = 10000
E = 160000
D = 128
DS = 16
DE = 16
H = 8
K = 2
WIDTH = 2 * D
MLP_IN = 2 * D + 2 * DS + DE
GW = 3 * D
HS = D // H

NCORE = 2
NSUB = 16
NWORK = NCORE * NSUB
CH = 128
EP = 163840
NCHUNK = EP // CH
CPW = NCHUNK // NWORK
NA = 10240
RPS = NA // NSUB
F32 = jnp.float32

def _mesh():
    return plsc.VectorSubcoreMesh(core_axis_name="c", subcore_axis_name="s",
                                  num_cores=NCORE, num_subcores=NSUB)


def _zero_vmem(buf, cols):
    z = jnp.zeros((16,), F32)

    @pl.loop(0, buf.shape[0])
    def _(i):
        for j in range(cols // 16):
            buf[i, pl.ds(j * 16, 16)] = z


NB = 4
NBS = 2


def _sc_gather_multi(tables, idxs):
    T = len(idxs)

    @functools.partial(
        pl.kernel,
        out_type=[jax.ShapeDtypeStruct((EP, D), F32) for _ in range(T)],
        mesh=_mesh(),
        scratch_types=[
            pltpu.VMEM((CPW, CH), jnp.int32),
            pltpu.VMEM((NB, CH, D), F32),
            pltpu.SemaphoreType.DMA,
        ],
    )
    def k(*refs):
        tabs = refs[:T]
        idx_h = refs[T:2 * T]
        outs = refs[2 * T:3 * T]
        slab, rows, sem = refs[3 * T:]
        wid = lax.axis_index("s") * NCORE + lax.axis_index("c")
        c0 = wid * CPW

        for t in range(T):
            pltpu.sync_copy(idx_h[t].at[pl.ds(c0, CPW)], slab)

            @pl.loop(0, CPW // NB)
            def _(jo):
                cs = [
                    pltpu.async_copy(
                        tabs[t].at[slab.at[jo * NB + b]], rows.at[b], sem)
                    for b in range(NB)
                ]
                for b in range(NB):
                    cs[b].wait()
                    pltpu.sync_copy(
                        rows.at[b],
                        outs[t].at[pl.ds((c0 + jo * NB + b) * CH, CH)])

    return k(*tables, *idxs)


def _sc_scatter_add(vals, idx):
    @functools.partial(
        pl.kernel,
        out_type=jax.ShapeDtypeStruct((NCORE * NA, D), F32),
        mesh=_mesh(),
        scratch_types=[
            pltpu.VMEM((CPW, CH), jnp.int32),
            pltpu.VMEM((NBS, CH, D), F32),
            pltpu.VMEM_SHARED((NA, D), F32),
            pltpu.SemaphoreType.DMA,
        ],
    )
    def k(vals_h, idx_h, out_o, slab, vbuf, acc, sem):
        cid = lax.axis_index("c")
        sid = lax.axis_index("s")
        c0 = cid * (NCHUNK // NCORE) + sid * CPW

        _zero_vmem(vbuf.at[0], D)
        for b in range(RPS // CH):
            pltpu.sync_copy(vbuf.at[0], acc.at[pl.ds(sid * RPS + b * CH, CH)])
        pltpu.sync_copy(idx_h.at[pl.ds(c0, CPW)], slab)
        plsc.subcore_barrier()

        @pl.loop(0, CPW // NBS)
        def _(jo):
            cs = [
                pltpu.async_copy(
                    vals_h.at[pl.ds((c0 + jo * NBS + b) * CH, CH)],
                    vbuf.at[b], sem)
                for b in range(NBS)
            ]
            for b in range(NBS):
                cs[b].wait()
                pltpu.sync_copy(vbuf.at[b], acc.at[slab.at[jo * NBS + b]],
                                add=True)

        plsc.subcore_barrier()
        for b in range(RPS // CH):
            r = sid * RPS + b * CH
            pltpu.sync_copy(acc.at[pl.ds(r, CH)], vbuf.at[0])
            pltpu.sync_copy(vbuf.at[0], out_o.at[pl.ds(cid * NA + r, CH)])

    return k(vals, idx).reshape(NCORE, NA, D)


CPW2 = NCHUNK // NSUB


def _sc_scatter_add2(vals0, vals1, idx):
    @functools.partial(
        pl.kernel,
        out_type=jax.ShapeDtypeStruct((NCORE * NA, D), F32),
        mesh=_mesh(),
        scratch_types=[
            pltpu.VMEM((CPW2, CH), jnp.int32),
            pltpu.VMEM((NBS, CH, D), F32),
            pltpu.VMEM_SHARED((NA, D), F32),
            pltpu.SemaphoreType.DMA,
        ],
    )
    def k(v0_h, v1_h, idx_h, out_o, slab, vbuf, acc, sem):
        cid = lax.axis_index("c")
        sid = lax.axis_index("s")
        c0 = sid * CPW2

        _zero_vmem(vbuf.at[0], D)
        for b in range(RPS // CH):
            pltpu.sync_copy(vbuf.at[0], acc.at[pl.ds(sid * RPS + b * CH, CH)])
        pltpu.sync_copy(idx_h.at[pl.ds(c0, CPW2)], slab)
        plsc.subcore_barrier()

        def edge_loop(vals_h):
            @pl.loop(0, CPW2 // NBS)
            def _(jo):
                cs = [
                    pltpu.async_copy(
                        vals_h.at[pl.ds((c0 + jo * NBS + b) * CH, CH)],
                        vbuf.at[b], sem)
                    for b in range(NBS)
                ]
                for b in range(NBS):
                    cs[b].wait()
                    pltpu.sync_copy(vbuf.at[b],
                                    acc.at[slab.at[jo * NBS + b]], add=True)

        @pl.when(cid == 0)
        def _():
            edge_loop(v0_h)

        @pl.when(cid == 1)
        def _():
            edge_loop(v1_h)

        plsc.subcore_barrier()
        for b in range(RPS // CH):
            r = sid * RPS + b * CH
            pltpu.sync_copy(acc.at[pl.ds(r, CH)], vbuf.at[0])
            pltpu.sync_copy(vbuf.at[0], out_o.at[pl.ds(cid * NA + r, CH)])

    out = k(vals0, vals1, idx)
    return out[:NA], out[NA:]


BN = 2000
BEB = 2048


def _dot16(a, b):
    return jnp.dot(a.astype(jnp.bfloat16), b.astype(jnp.bfloat16),
                   preferred_element_type=F32)


def _head_expand():
    lane = lax.broadcasted_iota(jnp.int32, (H, D), 1)
    row = lax.broadcasted_iota(jnp.int32, (H, D), 0)
    return (lane // HS == row).astype(F32)


def _ln_math(h, g, b):
    m = jnp.mean(h, axis=-1, keepdims=True)
    v = jnp.mean((h - m) ** 2, axis=-1, keepdims=True)
    return (h - m) * jax.lax.rsqrt(v + 1e-5) * g + b


def _tc_ln(h, lng, lnb):
    def body(g_ref, b_ref, h_ref, o_ref):
        o_ref[...] = _ln_math(h_ref[...], g_ref[...], b_ref[...])

    return pl.pallas_call(
        body,
        grid=(N // BN,),
        in_specs=[
            pl.BlockSpec((1, D), lambda i: (0, 0)),
            pl.BlockSpec((1, D), lambda i: (0, 0)),
            pl.BlockSpec((BN, D), lambda i: (i, 0)),
        ],
        out_specs=pl.BlockSpec((BN, D), lambda i: (i, 0)),
        out_shape=jax.ShapeDtypeStruct((N, D), F32),
    )(lng, lnb, h)


def _tc_edge_mlp(fwd, hs, hd, xsa, xsb, ef, at, bt, c1, c2, c3, w2t):
    def body(hs_ref, hd_ref, xsa_ref, xsb_ref, ef_ref,
             at_ref, bt_ref, c1_ref, c2_ref, c3_ref, w2_ref,
             gate_ref, msg_ref):
        hsv = hs_ref[...]
        pre = _dot16(hsv, at_ref[...])
        pre += _dot16(hd_ref[...], bt_ref[...])
        pre += _dot16(xsa_ref[...], c1_ref[...])
        pre += _dot16(xsb_ref[...], c2_ref[...])
        pre += _dot16(ef_ref[...], c3_ref[...])
        hid = jnp.maximum(pre, 0.0)
        raw = _dot16(hid, w2_ref[...])
        if fwd:
            raw = jnp.where(raw >= 0.0, raw, 0.01 * raw)
            gate = jnp.exp(raw * (1.0 / (HS ** 0.5)))
        else:
            gate = jax.nn.sigmoid(raw)
        gb = jnp.dot(gate[:, :H], _head_expand(),
                     preferred_element_type=F32,
                     precision=lax.Precision.HIGHEST)
        gate_ref[...] = gb if fwd else gate
        msg_ref[...] = hsv * gb

    gate_cols = D if fwd else 2 * H
    wspec = lambda shp: pl.BlockSpec(shp, lambda i: (0, 0))
    return pl.pallas_call(
        body,
        grid=(EP // BEB,),
        in_specs=[
            pl.BlockSpec((BEB, D), lambda i: (i, 0)),
            pl.BlockSpec((BEB, D), lambda i: (i, 0)),
            pl.BlockSpec((BEB, DS), lambda i: (i, 0)),
            pl.BlockSpec((BEB, DS), lambda i: (i, 0)),
            pl.BlockSpec((BEB, DE), lambda i: (i, 0)),
            wspec((D, WIDTH)), wspec((D, WIDTH)),
            wspec((DS, WIDTH)), wspec((DS, WIDTH)), wspec((DE, WIDTH)),
            wspec((WIDTH, 2 * H)),
        ],
        out_specs=[
            pl.BlockSpec((BEB, gate_cols), lambda i: (i, 0)),
            pl.BlockSpec((BEB, D), lambda i: (i, 0)),
        ],
        out_shape=[
            jax.ShapeDtypeStruct((EP, gate_cols), F32),
            jax.ShapeDtypeStruct((EP, D), F32),
        ],
    )(hs, hd, xsa, xsb, ef, at, bt, c1, c2, c3, w2t)


def _tc_edge_mlp2(hsf, hdf, hsr, hdr, xs_src, xs_dest, ef, wf, wr):
    def body(hsf_ref, hdf_ref, hsr_ref, hdr_ref, xss_ref, xsd_ref, ef_ref,
             fat, fbt, fc1, fc2, fc3, fw2,
             rat, rbt, rc1, rc2, rc3, rw2,
             gbf_ref, msgf_ref, gr_ref, msgr_ref):
        efv = ef_ref[...]

        def mlp(hs_v, hd_v, xa_v, xb_v, ws):
            at, bt, c1, c2, c3, w2 = ws
            pre = _dot16(hs_v, at[...])
            pre += _dot16(hd_v, bt[...])
            pre += _dot16(xa_v, c1[...])
            pre += _dot16(xb_v, c2[...])
            pre += _dot16(efv, c3[...])
            return _dot16(jnp.maximum(pre, 0.0), w2[...])

        hsfv = hsf_ref[...]
        raw = mlp(hsfv, hdf_ref[...], xss_ref[...], xsd_ref[...],
                  (fat, fbt, fc1, fc2, fc3, fw2))
        raw = jnp.where(raw >= 0.0, raw, 0.01 * raw)
        gate = jnp.exp(raw * (1.0 / (HS ** 0.5)))
        gb = jnp.dot(gate[:, :H], _head_expand(),
                     preferred_element_type=F32,
                     precision=lax.Precision.HIGHEST)
        gbf_ref[...] = gb
        msgf_ref[...] = hsfv * gb

        hsrv = hsr_ref[...]
        rraw = mlp(hsrv, hdr_ref[...], xsd_ref[...], xss_ref[...],
                   (rat, rbt, rc1, rc2, rc3, rw2))
        g = jax.nn.sigmoid(rraw)
        gr_ref[...] = g
        gbr = jnp.dot(g[:, :H], _head_expand(),
                      preferred_element_type=F32,
                      precision=lax.Precision.HIGHEST)
        msgr_ref[...] = hsrv * gbr

    espec = lambda c: pl.BlockSpec((BEB, c), lambda i: (i, 0))
    wspec = lambda shp: pl.BlockSpec(shp, lambda i: (0, 0))
    wspecs = [wspec((D, WIDTH)), wspec((D, WIDTH)), wspec((DS, WIDTH)),
              wspec((DS, WIDTH)), wspec((DE, WIDTH)), wspec((WIDTH, 2 * H))]
    return pl.pallas_call(
        body,
        grid=(EP // BEB,),
        in_specs=[espec(D)] * 4 + [espec(DS), espec(DS), espec(DE)]
        + wspecs + wspecs,
        out_specs=[espec(D), espec(D), espec(2 * H), espec(D)],
        out_shape=[
            jax.ShapeDtypeStruct((EP, D), F32),
            jax.ShapeDtypeStruct((EP, D), F32),
            jax.ShapeDtypeStruct((EP, 2 * H), F32),
            jax.ShapeDtypeStruct((EP, D), F32),
        ],
    )(hsf, hdf, hsr, hdr, xs_src, xs_dest, ef, *wf, *wr)


def _tc_update(h_prev, agg, pt, lng, lnb, smb=None):
    with_sm = smb is not None

    def body(*refs):
        if with_sm:
            g_ref, b_ref, h_ref, a_ref, s_ref, p_ref, hn_ref, ln_ref = refs
            agg_v = a_ref[...]
            smv = s_ref[...]
            agg_v = agg_v / jnp.where(smv > 0.0, smv, 1.0)
        else:
            g_ref, b_ref, h_ref, a_ref, p_ref, hn_ref, ln_ref = refs
            agg_v = a_ref[0] + a_ref[1]
        m = _dot16(agg_v, p_ref[...])
        hn = h_ref[...] + m
        hn_ref[...] = hn
        ln_ref[...] = _ln_math(hn, g_ref[...], b_ref[...])

    nspec = pl.BlockSpec((BN, D), lambda i: (i, 0))
    in_specs = [
        pl.BlockSpec((1, D), lambda i: (0, 0)),
        pl.BlockSpec((1, D), lambda i: (0, 0)),
        nspec,
    ]
    args = [lng, lnb, h_prev]
    if with_sm:
        in_specs += [nspec, nspec]
        args += [agg, smb]
    else:
        in_specs.append(pl.BlockSpec((NCORE, BN, D), lambda i: (0, i, 0)))
        args.append(agg)
    in_specs.append(pl.BlockSpec((D, D), lambda i: (0, 0)))
    args.append(pt)
    return pl.pallas_call(
        body,
        grid=(N // BN,),
        in_specs=in_specs,
        out_specs=[nspec, nspec],
        out_shape=[
            jax.ShapeDtypeStruct((N, D), F32),
            jax.ShapeDtypeStruct((N, D), F32),
        ],
    )(*args)


def _tc_wdiv(gb1, gb2, smg1, smg2):
    def body(e1, e2, s1, s2, w1, w2):
        comp = _head_expand().T * (1.0 / HS)
        for e, s_, w in ((e1, s1, w1), (e2, s2, w2)):
            wf = e[...] / (s_[...] + 1e-9)
            w[...] = jnp.dot(wf, comp, preferred_element_type=F32,
                             precision=lax.Precision.HIGHEST)

    return pl.pallas_call(
        body,
        grid=(EP // BEB,),
        in_specs=[pl.BlockSpec((BEB, D), lambda i: (i, 0))] * 4,
        out_specs=[pl.BlockSpec((BEB, H), lambda i: (i, 0))] * 2,
        out_shape=[jax.ShapeDtypeStruct((EP, H), F32)] * 2,
    )(gb1, gb2, smg1, smg2)


def _tc_gating(x, hf, hr, rw1t, rb1, rw2t, rb2, uw1t, ub1, uw2t, ub2,
               cw1t, cb1, cw2t, cb2):
    def body(x_ref, hf_ref, hr_ref,
             rw1_ref, rb1_ref, rw2_ref, rb2_ref,
             uw1_ref, ub1_ref, uw2_ref, ub2_ref,
             cw1_ref, cb1_ref, cw2_ref, cb2_ref,
             fin_ref, z_ref, r_ref):
        xv = x_ref[...]
        mf = hf_ref[...] - xv
        mr = hr_ref[...] - xv

        def mlp2(a0, w1_ref, b1_ref, w2_ref, b2_ref):
            h1 = _dot16(a0, w1_ref[pl.ds(0, D), :])
            h1 += _dot16(mf, w1_ref[pl.ds(D, D), :])
            h1 += _dot16(mr, w1_ref[pl.ds(2 * D, D), :])
            h1 = jnp.maximum(h1 + b1_ref[...], 0.0)
            return _dot16(h1, w2_ref[...]) + b2_ref[...]

        r = jax.nn.sigmoid(mlp2(xv, rw1_ref, rb1_ref, rw2_ref, rb2_ref))
        z = jax.nn.sigmoid(mlp2(xv, uw1_ref, ub1_ref, uw2_ref, ub2_ref))
        c = jnp.tanh(mlp2(r * xv, cw1_ref, cb1_ref, cw2_ref, cb2_ref))
        fin_ref[...] = (1.0 - z) * xv + z * c
        z_ref[...] = z
        r_ref[...] = r

    nspec = pl.BlockSpec((BN, D), lambda i: (i, 0))
    w1spec = pl.BlockSpec((GW, GW), lambda i: (0, 0))
    b1spec = pl.BlockSpec((1, GW), lambda i: (0, 0))
    w2spec = pl.BlockSpec((GW, D), lambda i: (0, 0))
    b2spec = pl.BlockSpec((1, D), lambda i: (0, 0))
    return pl.pallas_call(
        body,
        grid=(N // BN,),
        in_specs=[nspec, nspec, nspec] + [w1spec, b1spec, w2spec, b2spec] * 3,
        out_specs=[nspec, nspec, nspec],
        out_shape=[jax.ShapeDtypeStruct((N, D), F32)] * 3,
    )(x, hf, hr, rw1t, rb1, rw2t, rb2, uw1t, ub1, uw2t, ub2,
      cw1t, cb1, cw2t, cb2)


def _pad_idx(a, pad_val):
    pad = jnp.full((EP - E,), pad_val, jnp.int32)
    return jnp.concatenate([a, pad]).reshape(NCHUNK, CH)


def kernel(x, x_s, edge_index, edge_features, fw1, fw2, fp, rw1, rw2, rp,
           lng, lnb, rg_w1, rg_b1, rg_w2, rg_b2, ug_w1, ug_b1, ug_w2, ug_b2,
           cd_w1, cd_b1, cd_w2, cd_b2):
    src = edge_index[0]
    dest = edge_index[1]
    src_g = _pad_idx(src, 0)
    dest_g = _pad_idx(dest, 0)
    src_s = _pad_idx(src, N)
    dest_s = _pad_idx(dest, N)
    ef_p = jnp.concatenate(
        [edge_features, jnp.zeros((EP - E, DE), F32)], axis=0)

    lng2 = lng.reshape(1, D)
    lnb2 = lnb.reshape(1, D)

    fw1t = fw1.transpose(0, 2, 1)
    rw1t = rw1.transpose(0, 2, 1)
    zpad = jnp.zeros((K, WIDTH, H), F32)
    fw2t = jnp.concatenate([fw2.transpose(0, 2, 1), zpad], axis=-1)
    rw2t = jnp.concatenate([rw2.transpose(0, 2, 1), zpad], axis=-1)
    fpt = fp.transpose(0, 2, 1)
    rpt = rp.transpose(0, 2, 1)

    def wsplit(w1t, i):
        return (w1t[i, :D], w1t[i, D:2 * D], w1t[i, 2 * D:2 * D + DS],
                w1t[i, 2 * D + DS:2 * D + 2 * DS], w1t[i, 2 * D + 2 * DS:])

    ln0 = _tc_ln(x, lng2, lnb2)

    xs_pad = jnp.concatenate([x_s, jnp.zeros((N, D - DS), F32)], axis=1)
    gx_src, gx_dest, hs1, hd1 = _sc_gather_multi(
        [xs_pad, xs_pad, ln0, ln0], [src_g, dest_g, src_g, dest_g])
    xs_src = gx_src[:, :DS]
    xs_dest = gx_dest[:, :DS]

    gb1, msgf1 = _tc_edge_mlp(True, hs1, hd1, xs_src, xs_dest, ef_p,
                              *wsplit(fw1t, 0), fw2t[0])
    g1, msgr1 = _tc_edge_mlp(False, hd1, hs1, xs_dest, xs_src, ef_p,
                             *wsplit(rw1t, 0), rw2t[0])
    agg1, sm1 = _sc_scatter_add2(msgf1, gb1, dest_s)
    aggr1 = _sc_scatter_add(msgr1, src_s)
    h_f1, ln_f1 = _tc_update(x, agg1, fpt[0], lng2, lnb2, smb=sm1)
    h_r1, ln_r1 = _tc_update(x, aggr1, rpt[0], lng2, lnb2)

    hs2, hd2 = _sc_gather_multi([ln_f1, ln_f1], [src_g, dest_g])
    gb2, msgf2 = _tc_edge_mlp(True, hs2, hd2, xs_src, xs_dest, ef_p,
                              *wsplit(fw1t, 1), fw2t[1])
    hs2r, hd2r, smg1 = _sc_gather_multi(
        [ln_r1, ln_r1, sm1], [dest_g, src_g, dest_g])
    g2, msgr2 = _tc_edge_mlp(False, hs2r, hd2r, xs_dest, xs_src, ef_p,
                             *wsplit(rw1t, 1), rw2t[1])
    agg2, sm2 = _sc_scatter_add2(msgf2, gb2, dest_s)
    aggr2 = _sc_scatter_add(msgr2, src_s)
    h_f, _ = _tc_update(h_f1, agg2, fpt[1], lng2, lnb2, smb=sm2)
    h_r, _ = _tc_update(h_r1, aggr2, rpt[1], lng2, lnb2)
    gs = [g1, g2]

    (smg2,) = _sc_gather_multi([sm2], [dest_g])
    w1, w2 = _tc_wdiv(gb1, gb2, smg1, smg2)

    final, z, r = _tc_gating(
        x, h_f, h_r,
        rg_w1.T, rg_b1.reshape(1, GW), rg_w2.T, rg_b2.reshape(1, D),
        ug_w1.T, ug_b1.reshape(1, GW), ug_w2.T, ug_b2.reshape(1, D),
        cd_w1.T, cd_b1.reshape(1, GW), cd_w2.T, cd_b2.reshape(1, D))

    fws = jnp.stack([w1[:E], w2[:E]], axis=-1)
    rws = jnp.stack([gs[0][:E, :H], gs[1][:E, :H]], axis=-1)
    return final, fws, rws, z, r


# speedup vs baseline: 1.1757x; 1.0815x over previous
import functools

import jax
import jax.numpy as jnp
from jax import lax
from jax.experimental import pallas as pl
from jax.experimental.pallas import tpu as pltpu
from jax.experimental.pallas import tpu_sc as plsc

N = 10000
E = 160000
D = 128
DS = 16
DE = 16
H = 8
K = 2
WIDTH = 2 * D
MLP_IN = 2 * D + 2 * DS + DE
GW = 3 * D
HS = D // H

NCORE = 2
NSUB = 16
NWORK = NCORE * NSUB
CH = 128
EP = 163840
NCHUNK = EP // CH
CPW = NCHUNK // NWORK
NA = 10240
RPS = NA // NSUB
F32 = jnp.float32

def _mesh():
    return plsc.VectorSubcoreMesh(core_axis_name="c", subcore_axis_name="s",
                                  num_cores=NCORE, num_subcores=NSUB)


def _zero_vmem(buf, cols):
    z = jnp.zeros((16,), F32)

    @pl.loop(0, buf.shape[0])
    def _(i):
        for j in range(cols // 16):
            buf[i, pl.ds(j * 16, 16)] = z


NB = 6
NBS = 2


def _sc_gather_multi(tables, idxs):
    T = len(idxs)

    @functools.partial(
        pl.kernel,
        out_type=[jax.ShapeDtypeStruct((EP, D), F32) for _ in range(T)],
        mesh=_mesh(),
        scratch_types=[
            pltpu.VMEM((CPW, CH), jnp.int32),
            pltpu.VMEM((NB, CH, D), F32),
            pltpu.SemaphoreType.DMA,
        ],
    )
    def k(*refs):
        tabs = refs[:T]
        idx_h = refs[T:2 * T]
        outs = refs[2 * T:3 * T]
        slab, rows, sem = refs[3 * T:]
        wid = lax.axis_index("s") * NCORE + lax.axis_index("c")
        c0 = wid * CPW

        for t in range(T):
            pltpu.sync_copy(idx_h[t].at[pl.ds(c0, CPW)], slab)

            @pl.loop(0, CPW // NB)
            def _(jo):
                cs = [
                    pltpu.async_copy(
                        tabs[t].at[slab.at[jo * NB + b]], rows.at[b], sem)
                    for b in range(NB)
                ]
                for b in range(NB):
                    cs[b].wait()
                    pltpu.sync_copy(
                        rows.at[b],
                        outs[t].at[pl.ds((c0 + jo * NB + b) * CH, CH)])

    return k(*tables, *idxs)


def _sc_scatter_add(vals, idx):
    @functools.partial(
        pl.kernel,
        out_type=jax.ShapeDtypeStruct((NCORE * NA, D), F32),
        mesh=_mesh(),
        scratch_types=[
            pltpu.VMEM((CPW, CH), jnp.int32),
            pltpu.VMEM((NBS, CH, D), F32),
            pltpu.VMEM_SHARED((NA, D), F32),
            pltpu.SemaphoreType.DMA,
        ],
    )
    def k(vals_h, idx_h, out_o, slab, vbuf, acc, sem):
        cid = lax.axis_index("c")
        sid = lax.axis_index("s")
        c0 = cid * (NCHUNK // NCORE) + sid * CPW

        _zero_vmem(vbuf.at[0], D)
        for b in range(RPS // CH):
            pltpu.sync_copy(vbuf.at[0], acc.at[pl.ds(sid * RPS + b * CH, CH)])
        pltpu.sync_copy(idx_h.at[pl.ds(c0, CPW)], slab)
        plsc.subcore_barrier()

        @pl.loop(0, CPW // NBS)
        def _(jo):
            cs = [
                pltpu.async_copy(
                    vals_h.at[pl.ds((c0 + jo * NBS + b) * CH, CH)],
                    vbuf.at[b], sem)
                for b in range(NBS)
            ]
            for b in range(NBS):
                cs[b].wait()
                pltpu.sync_copy(vbuf.at[b], acc.at[slab.at[jo * NBS + b]],
                                add=True)

        plsc.subcore_barrier()
        for b in range(RPS // CH):
            r = sid * RPS + b * CH
            pltpu.sync_copy(acc.at[pl.ds(r, CH)], vbuf.at[0])
            pltpu.sync_copy(vbuf.at[0], out_o.at[pl.ds(cid * NA + r, CH)])

    return k(vals, idx).reshape(NCORE, NA, D)


CPW2 = NCHUNK // NSUB


def _sc_scatter_add2(vals0, vals1, idx):
    @functools.partial(
        pl.kernel,
        out_type=jax.ShapeDtypeStruct((NCORE * NA, D), F32),
        mesh=_mesh(),
        scratch_types=[
            pltpu.VMEM((CPW2, CH), jnp.int32),
            pltpu.VMEM((NBS, CH, D), F32),
            pltpu.VMEM_SHARED((NA, D), F32),
            pltpu.SemaphoreType.DMA,
        ],
    )
    def k(v0_h, v1_h, idx_h, out_o, slab, vbuf, acc, sem):
        cid = lax.axis_index("c")
        sid = lax.axis_index("s")
        c0 = sid * CPW2

        _zero_vmem(vbuf.at[0], D)
        for b in range(RPS // CH):
            pltpu.sync_copy(vbuf.at[0], acc.at[pl.ds(sid * RPS + b * CH, CH)])
        pltpu.sync_copy(idx_h.at[pl.ds(c0, CPW2)], slab)
        plsc.subcore_barrier()

        def edge_loop(vals_h):
            @pl.loop(0, CPW2 // NBS)
            def _(jo):
                cs = [
                    pltpu.async_copy(
                        vals_h.at[pl.ds((c0 + jo * NBS + b) * CH, CH)],
                        vbuf.at[b], sem)
                    for b in range(NBS)
                ]
                for b in range(NBS):
                    cs[b].wait()
                    pltpu.sync_copy(vbuf.at[b],
                                    acc.at[slab.at[jo * NBS + b]], add=True)

        @pl.when(cid == 0)
        def _():
            edge_loop(v0_h)

        @pl.when(cid == 1)
        def _():
            edge_loop(v1_h)

        plsc.subcore_barrier()
        for b in range(RPS // CH):
            r = sid * RPS + b * CH
            pltpu.sync_copy(acc.at[pl.ds(r, CH)], vbuf.at[0])
            pltpu.sync_copy(vbuf.at[0], out_o.at[pl.ds(cid * NA + r, CH)])

    out = k(vals0, vals1, idx)
    return out[:NA], out[NA:]


BN = 2000
BEB = 2048


def _dot16(a, b):
    return jnp.dot(a.astype(jnp.bfloat16), b.astype(jnp.bfloat16),
                   preferred_element_type=F32)


def _head_expand():
    lane = lax.broadcasted_iota(jnp.int32, (H, D), 1)
    row = lax.broadcasted_iota(jnp.int32, (H, D), 0)
    return (lane // HS == row).astype(F32)


def _ln_math(h, g, b):
    m = jnp.mean(h, axis=-1, keepdims=True)
    v = jnp.mean((h - m) ** 2, axis=-1, keepdims=True)
    return (h - m) * jax.lax.rsqrt(v + 1e-5) * g + b


def _tc_ln(h, lng, lnb):
    def body(g_ref, b_ref, h_ref, o_ref):
        o_ref[...] = _ln_math(h_ref[...], g_ref[...], b_ref[...])

    return pl.pallas_call(
        body,
        grid=(N // BN,),
        in_specs=[
            pl.BlockSpec((1, D), lambda i: (0, 0)),
            pl.BlockSpec((1, D), lambda i: (0, 0)),
            pl.BlockSpec((BN, D), lambda i: (i, 0)),
        ],
        out_specs=pl.BlockSpec((BN, D), lambda i: (i, 0)),
        out_shape=jax.ShapeDtypeStruct((N, D), F32),
    )(lng, lnb, h)


def _tc_edge_mlp(fwd, hs, hd, xsa, xsb, ef, at, bt, c1, c2, c3, w2t):
    def body(hs_ref, hd_ref, xsa_ref, xsb_ref, ef_ref,
             at_ref, bt_ref, c1_ref, c2_ref, c3_ref, w2_ref,
             gate_ref, msg_ref):
        hsv = hs_ref[...]
        pre = _dot16(hsv, at_ref[...])
        pre += _dot16(hd_ref[...], bt_ref[...])
        pre += _dot16(xsa_ref[...], c1_ref[...])
        pre += _dot16(xsb_ref[...], c2_ref[...])
        pre += _dot16(ef_ref[...], c3_ref[...])
        hid = jnp.maximum(pre, 0.0)
        raw = _dot16(hid, w2_ref[...])
        if fwd:
            raw = jnp.where(raw >= 0.0, raw, 0.01 * raw)
            gate = jnp.exp(raw * (1.0 / (HS ** 0.5)))
        else:
            gate = jax.nn.sigmoid(raw)
        gb = jnp.dot(gate[:, :H], _head_expand(),
                     preferred_element_type=F32,
                     precision=lax.Precision.HIGHEST)
        gate_ref[...] = gb if fwd else gate
        msg_ref[...] = hsv * gb

    gate_cols = D if fwd else 2 * H
    wspec = lambda shp: pl.BlockSpec(shp, lambda i: (0, 0))
    return pl.pallas_call(
        body,
        grid=(EP // BEB,),
        in_specs=[
            pl.BlockSpec((BEB, D), lambda i: (i, 0)),
            pl.BlockSpec((BEB, D), lambda i: (i, 0)),
            pl.BlockSpec((BEB, DS), lambda i: (i, 0)),
            pl.BlockSpec((BEB, DS), lambda i: (i, 0)),
            pl.BlockSpec((BEB, DE), lambda i: (i, 0)),
            wspec((D, WIDTH)), wspec((D, WIDTH)),
            wspec((DS, WIDTH)), wspec((DS, WIDTH)), wspec((DE, WIDTH)),
            wspec((WIDTH, 2 * H)),
        ],
        out_specs=[
            pl.BlockSpec((BEB, gate_cols), lambda i: (i, 0)),
            pl.BlockSpec((BEB, D), lambda i: (i, 0)),
        ],
        out_shape=[
            jax.ShapeDtypeStruct((EP, gate_cols), F32),
            jax.ShapeDtypeStruct((EP, D), F32),
        ],
    )(hs, hd, xsa, xsb, ef, at, bt, c1, c2, c3, w2t)


def _tc_edge_mlp2(hsf, hdf, hsr, hdr, xs_src, xs_dest, ef, wf, wr):
    def body(hsf_ref, hdf_ref, hsr_ref, hdr_ref, xss_ref, xsd_ref, ef_ref,
             fat, fbt, fc1, fc2, fc3, fw2,
             rat, rbt, rc1, rc2, rc3, rw2,
             gbf_ref, msgf_ref, gr_ref, msgr_ref):
        efv = ef_ref[...]

        def mlp(hs_v, hd_v, xa_v, xb_v, ws):
            at, bt, c1, c2, c3, w2 = ws
            pre = _dot16(hs_v, at[...])
            pre += _dot16(hd_v, bt[...])
            pre += _dot16(xa_v, c1[...])
            pre += _dot16(xb_v, c2[...])
            pre += _dot16(efv, c3[...])
            return _dot16(jnp.maximum(pre, 0.0), w2[...])

        hsfv = hsf_ref[...]
        raw = mlp(hsfv, hdf_ref[...], xss_ref[...], xsd_ref[...],
                  (fat, fbt, fc1, fc2, fc3, fw2))
        raw = jnp.where(raw >= 0.0, raw, 0.01 * raw)
        gate = jnp.exp(raw * (1.0 / (HS ** 0.5)))
        gb = jnp.dot(gate[:, :H], _head_expand(),
                     preferred_element_type=F32,
                     precision=lax.Precision.HIGHEST)
        gbf_ref[...] = gb
        msgf_ref[...] = hsfv * gb

        hsrv = hsr_ref[...]
        rraw = mlp(hsrv, hdr_ref[...], xsd_ref[...], xss_ref[...],
                   (rat, rbt, rc1, rc2, rc3, rw2))
        g = jax.nn.sigmoid(rraw)
        gr_ref[...] = g
        gbr = jnp.dot(g[:, :H], _head_expand(),
                      preferred_element_type=F32,
                      precision=lax.Precision.HIGHEST)
        msgr_ref[...] = hsrv * gbr

    espec = lambda c: pl.BlockSpec((BEB, c), lambda i: (i, 0))
    wspec = lambda shp: pl.BlockSpec(shp, lambda i: (0, 0))
    wspecs = [wspec((D, WIDTH)), wspec((D, WIDTH)), wspec((DS, WIDTH)),
              wspec((DS, WIDTH)), wspec((DE, WIDTH)), wspec((WIDTH, 2 * H))]
    return pl.pallas_call(
        body,
        grid=(EP // BEB,),
        in_specs=[espec(D)] * 4 + [espec(DS), espec(DS), espec(DE)]
        + wspecs + wspecs,
        out_specs=[espec(D), espec(D), espec(2 * H), espec(D)],
        out_shape=[
            jax.ShapeDtypeStruct((EP, D), F32),
            jax.ShapeDtypeStruct((EP, D), F32),
            jax.ShapeDtypeStruct((EP, 2 * H), F32),
            jax.ShapeDtypeStruct((EP, D), F32),
        ],
    )(hsf, hdf, hsr, hdr, xs_src, xs_dest, ef, *wf, *wr)


def _tc_update(h_prev, agg, pt, lng, lnb, smb=None):
    with_sm = smb is not None

    def body(*refs):
        if with_sm:
            g_ref, b_ref, h_ref, a_ref, s_ref, p_ref, hn_ref, ln_ref = refs
            agg_v = a_ref[...]
            smv = s_ref[...]
            agg_v = agg_v / jnp.where(smv > 0.0, smv, 1.0)
        else:
            g_ref, b_ref, h_ref, a_ref, p_ref, hn_ref, ln_ref = refs
            agg_v = a_ref[0] + a_ref[1]
        m = _dot16(agg_v, p_ref[...])
        hn = h_ref[...] + m
        hn_ref[...] = hn
        ln_ref[...] = _ln_math(hn, g_ref[...], b_ref[...])

    nspec = pl.BlockSpec((BN, D), lambda i: (i, 0))
    in_specs = [
        pl.BlockSpec((1, D), lambda i: (0, 0)),
        pl.BlockSpec((1, D), lambda i: (0, 0)),
        nspec,
    ]
    args = [lng, lnb, h_prev]
    if with_sm:
        in_specs += [nspec, nspec]
        args += [agg, smb]
    else:
        in_specs.append(pl.BlockSpec((NCORE, BN, D), lambda i: (0, i, 0)))
        args.append(agg)
    in_specs.append(pl.BlockSpec((D, D), lambda i: (0, 0)))
    args.append(pt)
    return pl.pallas_call(
        body,
        grid=(N // BN,),
        in_specs=in_specs,
        out_specs=[nspec, nspec],
        out_shape=[
            jax.ShapeDtypeStruct((N, D), F32),
            jax.ShapeDtypeStruct((N, D), F32),
        ],
    )(*args)


def _tc_wdiv(gb1, gb2, smg1, smg2):
    def body(e1, e2, s1, s2, w1, w2):
        comp = _head_expand().T * (1.0 / HS)
        for e, s_, w in ((e1, s1, w1), (e2, s2, w2)):
            wf = e[...] / (s_[...] + 1e-9)
            w[...] = jnp.dot(wf, comp, preferred_element_type=F32,
                             precision=lax.Precision.HIGHEST)

    return pl.pallas_call(
        body,
        grid=(EP // BEB,),
        in_specs=[pl.BlockSpec((BEB, D), lambda i: (i, 0))] * 4,
        out_specs=[pl.BlockSpec((BEB, H), lambda i: (i, 0))] * 2,
        out_shape=[jax.ShapeDtypeStruct((EP, H), F32)] * 2,
    )(gb1, gb2, smg1, smg2)


def _tc_gating(x, hf, hr, rw1t, rb1, rw2t, rb2, uw1t, ub1, uw2t, ub2,
               cw1t, cb1, cw2t, cb2):
    def body(x_ref, hf_ref, hr_ref,
             rw1_ref, rb1_ref, rw2_ref, rb2_ref,
             uw1_ref, ub1_ref, uw2_ref, ub2_ref,
             cw1_ref, cb1_ref, cw2_ref, cb2_ref,
             fin_ref, z_ref, r_ref):
        xv = x_ref[...]
        mf = hf_ref[...] - xv
        mr = hr_ref[...] - xv

        def mlp2(a0, w1_ref, b1_ref, w2_ref, b2_ref):
            h1 = _dot16(a0, w1_ref[pl.ds(0, D), :])
            h1 += _dot16(mf, w1_ref[pl.ds(D, D), :])
            h1 += _dot16(mr, w1_ref[pl.ds(2 * D, D), :])
            h1 = jnp.maximum(h1 + b1_ref[...], 0.0)
            return _dot16(h1, w2_ref[...]) + b2_ref[...]

        r = jax.nn.sigmoid(mlp2(xv, rw1_ref, rb1_ref, rw2_ref, rb2_ref))
        z = jax.nn.sigmoid(mlp2(xv, uw1_ref, ub1_ref, uw2_ref, ub2_ref))
        c = jnp.tanh(mlp2(r * xv, cw1_ref, cb1_ref, cw2_ref, cb2_ref))
        fin_ref[...] = (1.0 - z) * xv + z * c
        z_ref[...] = z
        r_ref[...] = r

    nspec = pl.BlockSpec((BN, D), lambda i: (i, 0))
    w1spec = pl.BlockSpec((GW, GW), lambda i: (0, 0))
    b1spec = pl.BlockSpec((1, GW), lambda i: (0, 0))
    w2spec = pl.BlockSpec((GW, D), lambda i: (0, 0))
    b2spec = pl.BlockSpec((1, D), lambda i: (0, 0))
    return pl.pallas_call(
        body,
        grid=(N // BN,),
        in_specs=[nspec, nspec, nspec] + [w1spec, b1spec, w2spec, b2spec] * 3,
        out_specs=[nspec, nspec, nspec],
        out_shape=[jax.ShapeDtypeStruct((N, D), F32)] * 3,
    )(x, hf, hr, rw1t, rb1, rw2t, rb2, uw1t, ub1, uw2t, ub2,
      cw1t, cb1, cw2t, cb2)


def _pad_idx(a, pad_val):
    pad = jnp.full((EP - E,), pad_val, jnp.int32)
    return jnp.concatenate([a, pad]).reshape(NCHUNK, CH)


def kernel(x, x_s, edge_index, edge_features, fw1, fw2, fp, rw1, rw2, rp,
           lng, lnb, rg_w1, rg_b1, rg_w2, rg_b2, ug_w1, ug_b1, ug_w2, ug_b2,
           cd_w1, cd_b1, cd_w2, cd_b2):
    src = edge_index[0]
    dest = edge_index[1]
    src_g = _pad_idx(src, 0)
    dest_g = _pad_idx(dest, 0)
    src_s = _pad_idx(src, N)
    dest_s = _pad_idx(dest, N)
    ef_p = jnp.concatenate(
        [edge_features, jnp.zeros((EP - E, DE), F32)], axis=0)

    lng2 = lng.reshape(1, D)
    lnb2 = lnb.reshape(1, D)

    fw1t = fw1.transpose(0, 2, 1)
    rw1t = rw1.transpose(0, 2, 1)
    zpad = jnp.zeros((K, WIDTH, H), F32)
    fw2t = jnp.concatenate([fw2.transpose(0, 2, 1), zpad], axis=-1)
    rw2t = jnp.concatenate([rw2.transpose(0, 2, 1), zpad], axis=-1)
    fpt = fp.transpose(0, 2, 1)
    rpt = rp.transpose(0, 2, 1)

    def wsplit(w1t, i):
        return (w1t[i, :D], w1t[i, D:2 * D], w1t[i, 2 * D:2 * D + DS],
                w1t[i, 2 * D + DS:2 * D + 2 * DS], w1t[i, 2 * D + 2 * DS:])

    xs_pad = jnp.concatenate([x_s, jnp.zeros((N, D - DS), F32)], axis=1)
    gx_src, gx_dest = _sc_gather_multi([xs_pad, xs_pad], [src_g, dest_g])
    xs_src = gx_src[:, :DS]
    xs_dest = gx_dest[:, :DS]

    ln0 = _tc_ln(x, lng2, lnb2)
    hs1, hd1 = _sc_gather_multi([ln0, ln0], [src_g, dest_g])

    gb1, msgf1 = _tc_edge_mlp(True, hs1, hd1, xs_src, xs_dest, ef_p,
                              *wsplit(fw1t, 0), fw2t[0])
    g1, msgr1 = _tc_edge_mlp(False, hd1, hs1, xs_dest, xs_src, ef_p,
                             *wsplit(rw1t, 0), rw2t[0])
    agg1, sm1 = _sc_scatter_add2(msgf1, gb1, dest_s)
    aggr1 = _sc_scatter_add(msgr1, src_s)
    h_f1, ln_f1 = _tc_update(x, agg1, fpt[0], lng2, lnb2, smb=sm1)
    h_r1, ln_r1 = _tc_update(x, aggr1, rpt[0], lng2, lnb2)

    hs2, hd2 = _sc_gather_multi([ln_f1, ln_f1], [src_g, dest_g])
    gb2, msgf2 = _tc_edge_mlp(True, hs2, hd2, xs_src, xs_dest, ef_p,
                              *wsplit(fw1t, 1), fw2t[1])
    hs2r, hd2r, smg1 = _sc_gather_multi(
        [ln_r1, ln_r1, sm1], [dest_g, src_g, dest_g])
    g2, msgr2 = _tc_edge_mlp(False, hs2r, hd2r, xs_dest, xs_src, ef_p,
                             *wsplit(rw1t, 1), rw2t[1])
    agg2, sm2 = _sc_scatter_add2(msgf2, gb2, dest_s)
    aggr2 = _sc_scatter_add(msgr2, src_s)
    h_f, _ = _tc_update(h_f1, agg2, fpt[1], lng2, lnb2, smb=sm2)
    h_r, _ = _tc_update(h_r1, aggr2, rpt[1], lng2, lnb2)
    gs = [g1, g2]

    (smg2,) = _sc_gather_multi([sm2], [dest_g])
    w1, w2 = _tc_wdiv(gb1, gb2, smg1, smg2)

    final, z, r = _tc_gating(
        x, h_f, h_r,
        rg_w1.T, rg_b1.reshape(1, GW), rg_w2.T, rg_b2.reshape(1, D),
        ug_w1.T, ug_b1.reshape(1, GW), ug_w2.T, ug_b2.reshape(1, D),
        cd_w1.T, cd_b1.reshape(1, GW), cd_w2.T, cd_b2.reshape(1, D))

    fws = jnp.stack([w1[:E], w2[:E]], axis=-1)
    rws = jnp.stack([gs[0][:E, :H], gs[1][:E, :H]], axis=-1)
    return final, fws, rws, z, r
